# baseline (speedup 1.0000x reference)
import jax
import jax.numpy as jnp
from jax.experimental import pallas as pl
from jax.experimental.pallas import tpu as pltpu


_TAPS2 = [(a, b) for a in (0, 1) for b in (0, 1)]
_TAPS3 = [(a, b) for a in (0, 1, 2) for b in (0, 1, 2)]
_TAPS4 = [(a, b) for a in (0, 1, 2, 3) for b in (0, 1, 2, 3)]

_BF = jnp.bfloat16


def _pad1(v):
    return jnp.pad(v, ((1, 1), (1, 1), (0, 0)))


def _mm(x, w):
    return jax.lax.dot_general(x, w, (((1,), (0,)), ((), ())),
                               preferred_element_type=jnp.float32)


def _conv(v, w, bvec, taps, H, W, relu):
    C = v.shape[-1]
    parts = [jax.lax.slice(v, (dy, dx, 0), (dy + H, dx + W, C))
             for (dy, dx) in taps]
    xc = jnp.concatenate(parts, axis=-1).reshape(H * W, len(taps) * C)
    acc = _mm(xc, w) + bvec
    if relu:
        acc = jnp.maximum(acc, 0.0)
    return acc.reshape(H, W, w.shape[1])


def _vq_chunk(zc, cbt16, cb16, cb2):
    scores = _mm(zc, cbt16)
    d2 = cb2 - 2.0 * scores
    dmin = jnp.min(d2, axis=1, keepdims=True)
    iota = jax.lax.broadcasted_iota(jnp.int32, d2.shape, 1)
    masked = jnp.where(d2 <= dmin, iota, d2.shape[1])
    idx = jnp.min(masked, axis=1, keepdims=True)
    onehot = (iota == idx).astype(jnp.float32)
    return _mm(onehot, cb16)


def _body(xs_ref, w1_ref, b1_ref, w2_ref, b2_ref, w3_ref, b3_ref,
          cbt_ref, cb_ref, wd1_ref, bd1_ref, wd2_ref, bd2_ref,
          wd3_ref, bd3_ref, o_ref, scr_ref):
    h = _conv(xs_ref[...], w1_ref[...], b1_ref[...], _TAPS2,
              112, 112, True)
    scr_ref[1:113, 1:113, :] = h
    scr_ref[0:1, :, :] = jnp.zeros((1, 114, 32), jnp.float32)
    scr_ref[113:114, :, :] = jnp.zeros((1, 114, 32), jnp.float32)
    scr_ref[1:113, 0:1, :] = jnp.zeros((112, 1, 32), jnp.float32)
    scr_ref[1:113, 113:114, :] = jnp.zeros((112, 1, 32), jnp.float32)
    parts = [scr_ref[pl.Slice(dy, 56, 2), pl.Slice(dx, 56, 2), :]
             for (dy, dx) in _TAPS4]
    xc = jnp.concatenate(parts, axis=-1).reshape(56 * 56, 16 * 32)
    h = jnp.maximum(_mm(xc, w2_ref[...]) + b2_ref[...], 0.0)
    h = h.reshape(56, 56, 64)
    z = _conv(_pad1(h), w3_ref[...], b3_ref[...], _TAPS3,
              56, 56, False)

    cbt = cbt_ref[...]
    cb2 = jnp.sum(cbt * cbt, axis=0, keepdims=True)
    cb16 = cb_ref[...]
    zf = z.reshape(3136, 64)
    qs = []
    m = 392
    for ci in range(8):
        zc = jax.lax.slice(zf, (ci * m, 0), ((ci + 1) * m, 64))
        qs.append(_vq_chunk(zc, cbt, cb16, cb2))
    q3 = jnp.concatenate(qs, axis=0).reshape(56, 56, 64)

    h = _conv(_pad1(q3), wd1_ref[...], bd1_ref[...], _TAPS3,
              56, 56, True)

    acc = _conv(_pad1(h), wd2_ref[...], bd2_ref[...], _TAPS3,
                56, 56, True)
    for p, (r, s) in enumerate([(0, 0), (0, 1), (1, 0), (1, 1)]):
        ph = jax.lax.slice(acc, (0, 0, p * 32), (56, 56, (p + 1) * 32))
        scr_ref[pl.Slice(1 + r, 56, 2), pl.Slice(1 + s, 56, 2), :] = ph

    for c0 in (0, 56):
        acc = _conv(scr_ref[pl.ds(c0, 58), :, :], wd3_ref[...], bd3_ref[...],
                    _TAPS3, 56, 112, False)
        o_ref[pl.ds(c0, 56), :, :] = acc


def _w_s2d(w):
    O, C = w.shape[0], w.shape[1]
    wt = w.transpose(2, 3, 1, 0)
    wt = wt.reshape(2, 2, 2, 2, C, O)
    wt = wt.transpose(0, 2, 1, 3, 4, 5)
    return wt.reshape(4 * 4 * C, O)


def _w_taps(w):
    O, C, kh, kw = w.shape
    return w.transpose(2, 3, 1, 0).reshape(kh * kw * C, O)


def _w_t_comb(w, opad=None):
    wt = jnp.transpose(w, (2, 3, 1, 0))
    if opad is not None and opad > wt.shape[-1]:
        wt = jnp.pad(wt, ((0, 0), (0, 0), (0, 0), (0, opad - wt.shape[-1])))
    I, O = wt.shape[2], wt.shape[3]
    comb = jnp.zeros((3, 3, I, 2, 2, O), wt.dtype)
    for t in range(3):
        for r in (0, 1):
            if not 0 <= t - r <= 1:
                continue
            for u in range(3):
                for s in (0, 1):
                    if not 0 <= u - s <= 1:
                        continue
                    comb = comb.at[t, u, :, r, s, :].set(wt[2 * t - r,
                                                            2 * u - s])
    return comb.reshape(9 * I, 4 * O)


def kernel(x, W1, b1, W2, b2, W3, b3, codebook, Wd1, bd1, Wd2, bd2, Wd3, bd3):
    B = x.shape[0]

    xh = x.transpose(0, 2, 3, 1)
    xh = jnp.pad(xh, ((0, 0), (1, 1), (1, 1), (0, 0)))
    xh = xh.reshape(B, 113, 2, 113, 2, 3).transpose(0, 1, 3, 2, 4, 5)
    xh = xh.reshape(B, 113, 113, 12)

    wd1e = jnp.transpose(jnp.flip(Wd1, axis=(2, 3)), (1, 0, 2, 3))
    wd2e = jnp.transpose(jnp.flip(Wd2, axis=(2, 3)), (1, 0, 2, 3))
    wd3e = jnp.transpose(jnp.flip(Wd3, axis=(2, 3)), (1, 0, 2, 3))

    args = (
        xh,
        _w_s2d(W1), b1.reshape(1, -1),
        _w_taps(W2), b2.reshape(1, -1),
        _w_taps(W3), b3.reshape(1, -1),
        codebook.T, codebook,
        _w_taps(wd1e), bd1.reshape(1, -1),
        _w_t_comb(wd2e), jnp.tile(bd2, 4).reshape(1, -1),
        _w_t_comb(wd3e, opad=8),
        jnp.tile(jnp.pad(bd3, (0, 5)), 4).reshape(1, -1),
    )

    def full(a):
        n = len(a.shape)
        return pl.BlockSpec(a.shape, lambda b, _n=n: (0,) * _n)

    in_specs = [pl.BlockSpec((None, 113, 113, 12), lambda b: (b, 0, 0, 0))]
    in_specs += [full(a) for a in args[1:]]

    out = pl.pallas_call(
        _body,
        grid=(B,),
        in_specs=in_specs,
        out_specs=pl.BlockSpec((None, 112, 112, 32), lambda b: (b, 0, 0, 0)),
        out_shape=jax.ShapeDtypeStruct((B, 112, 112, 32), jnp.float32),
        scratch_shapes=[pltpu.VMEM((114, 114, 32), jnp.float32)],
    )(*args)

    out = out.reshape(B, 112, 112, 2, 2, 8).transpose(0, 1, 3, 2, 4, 5)
    out = out.reshape(B, 224, 224, 8)[..., :3]
    return out.transpose(0, 3, 1, 2)


# speedup vs baseline: 1.0592x; 1.0592x over previous
import jax
import jax.numpy as jnp
from jax.experimental import pallas as pl
from jax.experimental.pallas import tpu as pltpu


_TAPS2 = [(a, b) for a in (0, 1) for b in (0, 1)]
_TAPS3 = [(a, b) for a in (0, 1, 2) for b in (0, 1, 2)]
_TAPS4 = [(a, b) for a in (0, 1, 2, 3) for b in (0, 1, 2, 3)]

_BF = jnp.bfloat16


def _pad1(v):
    return jnp.pad(v, ((1, 1), (1, 1), (0, 0)))


def _mm(x, w):
    return jax.lax.dot_general(x, w, (((1,), (0,)), ((), ())),
                               preferred_element_type=jnp.float32)


def _conv(v, w, bvec, taps, H, W, relu):
    C = v.shape[-1]
    parts = [jax.lax.slice(v, (dy, dx, 0), (dy + H, dx + W, C))
             for (dy, dx) in taps]
    xc = jnp.concatenate(parts, axis=-1).reshape(H * W, len(taps) * C)
    acc = _mm(xc, w) + bvec
    if relu:
        acc = jnp.maximum(acc, 0.0)
    return acc.reshape(H, W, w.shape[1])


def _vq_chunk(zc, cbt16, cb16, cb2):
    scores = _mm(zc, cbt16)
    d2 = cb2 - 2.0 * scores
    dmin = jnp.min(d2, axis=1, keepdims=True)
    iota = jax.lax.broadcasted_iota(jnp.int32, d2.shape, 1)
    masked = jnp.where(d2 <= dmin, iota, d2.shape[1])
    idx = jnp.min(masked, axis=1, keepdims=True)
    onehot = (iota == idx).astype(jnp.float32)
    return _mm(onehot, cb16)


def _body(xs_ref, w1_ref, b1_ref, w2_ref, b2_ref, w3_ref, b3_ref,
          cbt_ref, cb_ref, wd1_ref, bd1_ref, wd2_ref, bd2_ref,
          wd3_ref, bd3_ref, o_ref, scr_ref):
    h = _conv(xs_ref[...], w1_ref[...], b1_ref[...], _TAPS2,
              112, 112, True)
    scr_ref[1:113, 1:113, :] = h
    scr_ref[0:1, :, :] = jnp.zeros((1, 114, 32), jnp.float32)
    scr_ref[113:114, :, :] = jnp.zeros((1, 114, 32), jnp.float32)
    scr_ref[1:113, 0:1, :] = jnp.zeros((112, 1, 32), jnp.float32)
    scr_ref[1:113, 113:114, :] = jnp.zeros((112, 1, 32), jnp.float32)
    parts = [scr_ref[pl.Slice(dy, 56, 2), pl.Slice(dx, 56, 2), :]
             for (dy, dx) in _TAPS4]
    xc = jnp.concatenate(parts, axis=-1).reshape(56 * 56, 16 * 32)
    h = jnp.maximum(_mm(xc, w2_ref[...]) + b2_ref[...], 0.0)
    h = h.reshape(56, 56, 64)
    z = _conv(_pad1(h), w3_ref[...], b3_ref[...], _TAPS3,
              56, 56, False)

    cbt = cbt_ref[...]
    cb2 = jnp.sum(cbt * cbt, axis=0, keepdims=True)
    cb16 = cb_ref[...]
    zf = z.reshape(3136, 64)
    qs = []
    m = 392
    for ci in range(8):
        zc = jax.lax.slice(zf, (ci * m, 0), ((ci + 1) * m, 64))
        qs.append(_vq_chunk(zc, cbt, cb16, cb2))
    q3 = jnp.concatenate(qs, axis=0).reshape(56, 56, 64)

    h = _conv(_pad1(q3), wd1_ref[...], bd1_ref[...], _TAPS3,
              56, 56, True)

    acc = _conv(_pad1(h), wd2_ref[...], bd2_ref[...], _TAPS3,
                56, 56, True)
    for p, (r, s) in enumerate([(0, 0), (0, 1), (1, 0), (1, 1)]):
        ph = jax.lax.slice(acc, (0, 0, p * 32), (56, 56, (p + 1) * 32))
        scr_ref[pl.Slice(1 + r, 56, 2), pl.Slice(1 + s, 56, 2), :] = ph

    ix = jax.lax.broadcasted_iota(jnp.int32, (56, 112), 1)
    gidx = (ix % 2) * 56 + ix // 2
    for c0 in (0, 56):
        acc = _conv(scr_ref[pl.ds(c0, 58), :, :], wd3_ref[...], bd3_ref[...],
                    _TAPS3, 56, 112, False)
        for r in (0, 1):
            for c in range(3):
                l0 = (2 * r) * 8 + c
                l1 = (2 * r + 1) * 8 + c
                v0 = jax.lax.slice(acc, (0, 0, l0),
                                   (56, 112, l0 + 1)).reshape(56, 112)
                v1 = jax.lax.slice(acc, (0, 0, l1),
                                   (56, 112, l1 + 1)).reshape(56, 112)
                halves = []
                for hh in (0, 1):
                    cat = jnp.concatenate(
                        [jax.lax.slice(v0, (0, 56 * hh), (56, 56 * hh + 56)),
                         jax.lax.slice(v1, (0, 56 * hh), (56, 56 * hh + 56))],
                        axis=-1)
                    halves.append(jnp.take_along_axis(cat, gidx, axis=1))
                merged = jnp.concatenate(halves, axis=-1)
                o_ref[c:c + 1, pl.ds(c0, 56), r:r + 1, :] = (
                    merged.reshape(1, 56, 1, 224))


def _w_s2d(w):
    O, C = w.shape[0], w.shape[1]
    wt = w.transpose(2, 3, 1, 0)
    wt = wt.reshape(2, 2, 2, 2, C, O)
    wt = wt.transpose(0, 2, 1, 3, 4, 5)
    return wt.reshape(4 * 4 * C, O)


def _w_taps(w):
    O, C, kh, kw = w.shape
    return w.transpose(2, 3, 1, 0).reshape(kh * kw * C, O)


def _w_t_comb(w, opad=None):
    wt = jnp.transpose(w, (2, 3, 1, 0))
    if opad is not None and opad > wt.shape[-1]:
        wt = jnp.pad(wt, ((0, 0), (0, 0), (0, 0), (0, opad - wt.shape[-1])))
    I, O = wt.shape[2], wt.shape[3]
    comb = jnp.zeros((3, 3, I, 2, 2, O), wt.dtype)
    for t in range(3):
        for r in (0, 1):
            if not 0 <= t - r <= 1:
                continue
            for u in range(3):
                for s in (0, 1):
                    if not 0 <= u - s <= 1:
                        continue
                    comb = comb.at[t, u, :, r, s, :].set(wt[2 * t - r,
                                                            2 * u - s])
    return comb.reshape(9 * I, 4 * O)


def kernel(x, W1, b1, W2, b2, W3, b3, codebook, Wd1, bd1, Wd2, bd2, Wd3, bd3):
    B = x.shape[0]

    xh = x.transpose(0, 2, 3, 1)
    xh = jnp.pad(xh, ((0, 0), (1, 1), (1, 1), (0, 0)))
    xh = xh.reshape(B, 113, 2, 113, 2, 3).transpose(0, 1, 3, 2, 4, 5)
    xh = xh.reshape(B, 113, 113, 12)

    wd1e = jnp.transpose(jnp.flip(Wd1, axis=(2, 3)), (1, 0, 2, 3))
    wd2e = jnp.transpose(jnp.flip(Wd2, axis=(2, 3)), (1, 0, 2, 3))
    wd3e = jnp.transpose(jnp.flip(Wd3, axis=(2, 3)), (1, 0, 2, 3))

    args = (
        xh,
        _w_s2d(W1), b1.reshape(1, -1),
        _w_taps(W2), b2.reshape(1, -1),
        _w_taps(W3), b3.reshape(1, -1),
        codebook.T, codebook,
        _w_taps(wd1e), bd1.reshape(1, -1),
        _w_t_comb(wd2e), jnp.tile(bd2, 4).reshape(1, -1),
        _w_t_comb(wd3e, opad=8),
        jnp.tile(jnp.pad(bd3, (0, 5)), 4).reshape(1, -1),
    )

    def full(a):
        n = len(a.shape)
        return pl.BlockSpec(a.shape, lambda b, _n=n: (0,) * _n)

    in_specs = [pl.BlockSpec((None, 113, 113, 12), lambda b: (b, 0, 0, 0))]
    in_specs += [full(a) for a in args[1:]]

    out = pl.pallas_call(
        _body,
        grid=(B,),
        in_specs=in_specs,
        out_specs=pl.BlockSpec((None, 3, 112, 2, 224),
                               lambda b: (b, 0, 0, 0, 0)),
        out_shape=jax.ShapeDtypeStruct((B, 3, 112, 2, 224), jnp.float32),
        scratch_shapes=[pltpu.VMEM((114, 114, 32), jnp.float32)],
    )(*args)

    return out.reshape(B, 3, 224, 224)


# speedup vs baseline: 1.0593x; 1.0001x over previous
import jax
import jax.numpy as jnp
from jax.experimental import pallas as pl
from jax.experimental.pallas import tpu as pltpu


_TAPS2 = [(a, b) for a in (0, 1) for b in (0, 1)]
_TAPS3 = [(a, b) for a in (0, 1, 2) for b in (0, 1, 2)]
_TAPS4 = [(a, b) for a in (0, 1, 2, 3) for b in (0, 1, 2, 3)]

_BF = jnp.bfloat16


def _pad1(v):
    return jnp.pad(v, ((1, 1), (1, 1), (0, 0)))


def _mm(x, w):
    return jax.lax.dot_general(x, w, (((1,), (0,)), ((), ())),
                               preferred_element_type=jnp.float32)


def _conv(v, w, bvec, taps, H, W, relu):
    C = v.shape[-1]
    parts = [jax.lax.slice(v, (dy, dx, 0), (dy + H, dx + W, C))
             for (dy, dx) in taps]
    xc = jnp.concatenate(parts, axis=-1).reshape(H * W, len(taps) * C)
    acc = _mm(xc, w) + bvec
    if relu:
        acc = jnp.maximum(acc, 0.0)
    return acc.reshape(H, W, w.shape[1])


def _vq_chunk(zc, cbt16, cb16, cb2):
    scores = _mm(zc, cbt16)
    d2 = cb2 - 2.0 * scores
    dmin = jnp.min(d2, axis=1, keepdims=True)
    iota = jax.lax.broadcasted_iota(jnp.int32, d2.shape, 1)
    masked = jnp.where(d2 <= dmin, iota, d2.shape[1])
    idx = jnp.min(masked, axis=1, keepdims=True)
    onehot = (iota == idx).astype(jnp.float32)
    return _mm(onehot, cb16)


def _body(xs_ref, w1_ref, b1_ref, w2_ref, b2_ref, w3_ref, b3_ref,
          cbt_ref, cb_ref, wd1_ref, bd1_ref, wd2_ref, bd2_ref,
          wd3_ref, bd3_ref, o_ref, scr_ref):
    h = _conv(xs_ref[...], w1_ref[...], b1_ref[...], _TAPS2,
              112, 112, True)
    scr_ref[1:113, 1:113, :] = h
    scr_ref[0:1, :, :] = jnp.zeros((1, 114, 32), jnp.float32)
    scr_ref[113:114, :, :] = jnp.zeros((1, 114, 32), jnp.float32)
    scr_ref[1:113, 0:1, :] = jnp.zeros((112, 1, 32), jnp.float32)
    scr_ref[1:113, 113:114, :] = jnp.zeros((112, 1, 32), jnp.float32)
    parts = [scr_ref[pl.Slice(dy, 56, 2), pl.Slice(dx, 56, 2), :]
             for (dy, dx) in _TAPS4]
    xc = jnp.concatenate(parts, axis=-1).reshape(56 * 56, 16 * 32)
    h = jnp.maximum(_mm(xc, w2_ref[...]) + b2_ref[...], 0.0)
    h = h.reshape(56, 56, 64)
    z = _conv(_pad1(h), w3_ref[...], b3_ref[...], _TAPS3,
              56, 56, False)

    cbt = cbt_ref[...]
    cb2 = jnp.sum(cbt * cbt, axis=0, keepdims=True)
    cb16 = cb_ref[...]
    zf = z.reshape(3136, 64)
    qs = []
    m = 392
    for ci in range(8):
        zc = jax.lax.slice(zf, (ci * m, 0), ((ci + 1) * m, 64))
        qs.append(_vq_chunk(zc, cbt, cb16, cb2))
    q3 = jnp.concatenate(qs, axis=0).reshape(56, 56, 64)

    h = _conv(_pad1(q3), wd1_ref[...], bd1_ref[...], _TAPS3,
              56, 56, True)

    acc = _conv(_pad1(h), wd2_ref[...], bd2_ref[...], _TAPS3,
                56, 56, True)
    for p, (r, s) in enumerate([(0, 0), (0, 1), (1, 0), (1, 1)]):
        ph = jax.lax.slice(acc, (0, 0, p * 32), (56, 56, (p + 1) * 32))
        scr_ref[pl.Slice(1 + r, 56, 2), pl.Slice(1 + s, 56, 2), :] = ph

    ix = jax.lax.broadcasted_iota(jnp.int32, (56, 112), 1)
    gidx = (ix % 2) * 56 + ix // 2
    for c0 in (0, 56):
        acc = _conv(scr_ref[pl.ds(c0, 58), :, :], wd3_ref[...], bd3_ref[...],
                    _TAPS3, 56, 112, False)
        for r in (0, 1):
            for c in range(3):
                l0 = (2 * r) * 8 + c
                l1 = (2 * r + 1) * 8 + c
                v0 = jax.lax.slice(acc, (0, 0, l0),
                                   (56, 112, l0 + 1)).reshape(56, 112)
                v1 = jax.lax.slice(acc, (0, 0, l1),
                                   (56, 112, l1 + 1)).reshape(56, 112)
                halves = []
                for hh in (0, 1):
                    cat = jnp.concatenate(
                        [jax.lax.slice(v0, (0, 56 * hh), (56, 56 * hh + 56)),
                         jax.lax.slice(v1, (0, 56 * hh), (56, 56 * hh + 56))],
                        axis=-1)
                    halves.append(jnp.take_along_axis(cat, gidx, axis=1))
                merged = jnp.concatenate(halves, axis=-1)
                o_ref[c:c + 1, pl.ds(c0, 56), r:r + 1, :] = (
                    merged.reshape(1, 56, 1, 224))


def _w_s2d(w):
    O, C = w.shape[0], w.shape[1]
    wt = w.transpose(2, 3, 1, 0)
    wt = wt.reshape(2, 2, 2, 2, C, O)
    wt = wt.transpose(0, 2, 1, 3, 4, 5)
    return wt.reshape(4 * 4 * C, O)


def _w_taps(w):
    O, C, kh, kw = w.shape
    return w.transpose(2, 3, 1, 0).reshape(kh * kw * C, O)


def _w_t_comb(w, opad=None):
    wt = jnp.transpose(w, (2, 3, 1, 0))
    if opad is not None and opad > wt.shape[-1]:
        wt = jnp.pad(wt, ((0, 0), (0, 0), (0, 0), (0, opad - wt.shape[-1])))
    I, O = wt.shape[2], wt.shape[3]
    comb = jnp.zeros((3, 3, I, 2, 2, O), wt.dtype)
    for t in range(3):
        for r in (0, 1):
            if not 0 <= t - r <= 1:
                continue
            for u in range(3):
                for s in (0, 1):
                    if not 0 <= u - s <= 1:
                        continue
                    comb = comb.at[t, u, :, r, s, :].set(wt[2 * t - r,
                                                            2 * u - s])
    return comb.reshape(9 * I, 4 * O)


def kernel(x, W1, b1, W2, b2, W3, b3, codebook, Wd1, bd1, Wd2, bd2, Wd3, bd3):
    B = x.shape[0]

    xp = jnp.pad(x, ((0, 0), (0, 0), (1, 1), (1, 1)))
    xp = xp.reshape(B, 3, 113, 2, 113, 2)
    xh = xp.transpose(0, 2, 4, 3, 5, 1).reshape(B, 113, 113, 12)

    wd1e = jnp.transpose(jnp.flip(Wd1, axis=(2, 3)), (1, 0, 2, 3))
    wd2e = jnp.transpose(jnp.flip(Wd2, axis=(2, 3)), (1, 0, 2, 3))
    wd3e = jnp.transpose(jnp.flip(Wd3, axis=(2, 3)), (1, 0, 2, 3))

    args = (
        xh,
        _w_s2d(W1), b1.reshape(1, -1),
        _w_taps(W2), b2.reshape(1, -1),
        _w_taps(W3), b3.reshape(1, -1),
        codebook.T, codebook,
        _w_taps(wd1e), bd1.reshape(1, -1),
        _w_t_comb(wd2e), jnp.tile(bd2, 4).reshape(1, -1),
        _w_t_comb(wd3e, opad=8),
        jnp.tile(jnp.pad(bd3, (0, 5)), 4).reshape(1, -1),
    )

    def full(a):
        n = len(a.shape)
        return pl.BlockSpec(a.shape, lambda b, _n=n: (0,) * _n)

    in_specs = [pl.BlockSpec((None, 113, 113, 12), lambda b: (b, 0, 0, 0))]
    in_specs += [full(a) for a in args[1:]]

    out = pl.pallas_call(
        _body,
        grid=(B,),
        in_specs=in_specs,
        out_specs=pl.BlockSpec((None, 3, 112, 2, 224),
                               lambda b: (b, 0, 0, 0, 0)),
        out_shape=jax.ShapeDtypeStruct((B, 3, 112, 2, 224), jnp.float32),
        scratch_shapes=[pltpu.VMEM((114, 114, 32), jnp.float32)],
    )(*args)

    return out.reshape(B, 3, 224, 224)
